# baseline (speedup 1.0000x reference)
import functools

import jax
import jax.numpy as jnp
from jax import lax
from jax.experimental import pallas as pl
from jax.experimental.pallas import tpu as pltpu
from jax.experimental.pallas import tpu_sc as plsc

N = 10000
E = 320000
F_IN = 128
F_H = 128
F_C = 40
F_CP = 48

NC, NS = 2, 16
NW = NC * NS
EPW = E // NW
K = 80
NCHUNK = EPW // K
KA = 96
NF = 104
KT = EPW - NF * KA
RPS = N // NS

_MESH = dict(core_axis_name="c", subcore_axis_name="s",
             num_cores=NC, num_subcores=NS)


@functools.cache
def _make_agg(D):
    @functools.partial(
        pl.kernel,
        out_type=jax.ShapeDtypeStruct((NC, NS, RPS, D), jnp.float32),
        mesh=plsc.VectorSubcoreMesh(**_MESH),
        scratch_types=[
            pltpu.VMEM((EPW,), jnp.int32),
            pltpu.VMEM((EPW,), jnp.int32),
            pltpu.VMEM((KA, D), jnp.float32),
            pltpu.VMEM((KA, D), jnp.float32),
            pltpu.VMEM((KT, D), jnp.float32),
            pltpu.VMEM_SHARED((N, D), jnp.float32),
            pltpu.SemaphoreType.DMA,
            pltpu.SemaphoreType.DMA,
            pltpu.SemaphoreType.DMA,
            pltpu.SemaphoreType.DMA,
        ],
    )
    def agg(h_hbm, src_hbm, dst_hbm, z_hbm, out_hbm,
            srcv, dstv, rows_a, rows_b, rows_t, acc, gs_a, gs_b, ss_a, ss_b):
        c = lax.axis_index("c")
        s = lax.axis_index("s")
        wid = s * NC + c
        base = wid * EPW

        def fire_gather(t, rows, sem, size=KA):
            return pltpu.async_copy(
                h_hbm.at[srcv.at[pl.ds(t * KA, size)]], rows, sem)

        def wait_gather(t, rows, sem, size=KA):
            pltpu.make_async_copy(
                h_hbm.at[srcv.at[pl.ds(t * KA, size)]], rows, sem).wait()

        def fire_scatter(t, rows, sem, size=KA):
            return pltpu.async_copy(
                rows, acc.at[dstv.at[pl.ds(t * KA, size)]], sem, add=True)

        def wait_scatter(t, rows, sem, size=KA):
            pltpu.make_async_copy(
                rows, acc.at[dstv.at[pl.ds(t * KA, size)]], sem).wait()

        pltpu.sync_copy(src_hbm.at[pl.ds(base, EPW)], srcv)
        pltpu.sync_copy(dst_hbm.at[pl.ds(base, EPW)], dstv)
        fire_gather(0, rows_a, gs_a)
        pltpu.sync_copy(z_hbm.at[s], acc.at[pl.ds(s * RPS, RPS)])
        plsc.subcore_barrier()

        def pair(p, carry):
            t = 2 * p
            wait_gather(t, rows_a, gs_a)
            fire_scatter(t, rows_a, ss_a)

            @pl.when(p > 0)
            def _():
                wait_scatter(t - 1, rows_b, ss_b)

            fire_gather(t + 1, rows_b, gs_b)
            wait_scatter(t, rows_a, ss_a)
            fire_gather(t + 2, rows_a, gs_a)
            wait_gather(t + 1, rows_b, gs_b)
            fire_scatter(t + 1, rows_b, ss_b)
            return carry

        lax.fori_loop(0, NF // 2 - 1, pair, 0)
        t2 = NF - 2
        wait_gather(t2, rows_a, gs_a)
        fire_scatter(t2, rows_a, ss_a)
        wait_scatter(t2 - 1, rows_b, ss_b)
        fire_gather(t2 + 1, rows_b, gs_b)
        d_t = pltpu.async_copy(
            h_hbm.at[srcv.at[pl.ds(NF * KA, KT)]], rows_t, gs_a)
        wait_scatter(t2, rows_a, ss_a)
        wait_gather(t2 + 1, rows_b, gs_b)
        fire_scatter(t2 + 1, rows_b, ss_b)
        d_t.wait()
        pltpu.sync_copy(rows_t, acc.at[dstv.at[pl.ds(NF * KA, KT)]], add=True)
        wait_scatter(t2 + 1, rows_b, ss_b)
        plsc.subcore_barrier()
        pltpu.sync_copy(acc.at[pl.ds(s * RPS, RPS)], out_hbm.at[c, s])

    return agg


@functools.cache
def _make_deg():
    @functools.partial(
        pl.kernel,
        out_type=jax.ShapeDtypeStruct((NC, 2, NS, RPS, 16), jnp.float32),
        mesh=plsc.VectorSubcoreMesh(**_MESH),
        scratch_types=[
            pltpu.VMEM((EPW,), jnp.int32),
            pltpu.VMEM((EPW,), jnp.int32),
            pltpu.VMEM((K, 16), jnp.float32),
            pltpu.VMEM_SHARED((N, 16), jnp.float32),
            pltpu.VMEM_SHARED((N, 16), jnp.float32),
            pltpu.SemaphoreType.DMA,
        ],
        compiler_params=pltpu.CompilerParams(use_tc_tiling_on_sc=False),
    )
    def _deg(src_hbm, dst_hbm, z_hbm, ones_hbm, out_hbm,
             srcv, dstv, ones_v, acc_o, acc_i, sem):
        c = lax.axis_index("c")
        s = lax.axis_index("s")
        wid = s * NC + c
        base = wid * EPW
        pltpu.sync_copy(src_hbm.at[pl.ds(base, EPW)], srcv)
        pltpu.sync_copy(dst_hbm.at[pl.ds(base, EPW)], dstv)
        pltpu.sync_copy(ones_hbm, ones_v)
        pltpu.sync_copy(z_hbm.at[s], acc_o.at[pl.ds(s * RPS, RPS)])
        pltpu.sync_copy(z_hbm.at[s], acc_i.at[pl.ds(s * RPS, RPS)])
        plsc.subcore_barrier()

        def fire(t, carry):
            pltpu.async_copy(ones_v, acc_o.at[srcv.at[pl.ds(t * K, K)]], sem,
                             add=True)
            pltpu.async_copy(ones_v, acc_i.at[dstv.at[pl.ds(t * K, K)]], sem,
                             add=True)
            return carry

        lax.fori_loop(0, NCHUNK, fire, 0)

        def drain(t, carry):
            pltpu.make_async_copy(ones_v, acc_o.at[srcv.at[pl.ds(t * K, K)]],
                                  sem).wait()
            pltpu.make_async_copy(ones_v, acc_i.at[dstv.at[pl.ds(t * K, K)]],
                                  sem).wait()
            return carry

        lax.fori_loop(0, NCHUNK, drain, 0)
        plsc.subcore_barrier()
        pltpu.sync_copy(acc_o.at[pl.ds(s * RPS, RPS)], out_hbm.at[c, 0, s])
        pltpu.sync_copy(acc_i.at[pl.ds(s * RPS, RPS)], out_hbm.at[c, 1, s])

    return _deg


R = 400


def _norms(dg, which):
    d = dg[0, which, :, 0:1] + dg[1, which, :, 0:1]
    return lax.rsqrt(jnp.maximum(d, 1.0))


def _tc_h1(x, degs, W1):
    def body(x_ref, deg_ref, w_ref, o_ref):
        dg = deg_ref[...]
        n_out = _norms(dg, 0)
        o_ref[...] = jnp.dot(x_ref[...] * n_out, w_ref[...],
                             preferred_element_type=jnp.float32)

    return pl.pallas_call(
        body,
        grid=(N // R,),
        in_specs=[
            pl.BlockSpec((R, F_IN), lambda i: (i, 0)),
            pl.BlockSpec((2, 2, R, 16), lambda i: (0, 0, i, 0)),
            pl.BlockSpec((F_IN, F_H), lambda i: (0, 0)),
        ],
        out_specs=pl.BlockSpec((R, F_H), lambda i: (i, 0)),
        out_shape=jax.ShapeDtypeStruct((N, F_H), jnp.float32),
    )(x, degs, W1)


def _tc_mid(agg_parts, degs, b1):
    def body(a_ref, deg_ref, b_ref, o_ref):
        dg = deg_ref[...]
        n_out = _norms(dg, 0)
        n_in = _norms(dg, 1)
        aggv = a_ref[0] + a_ref[1]
        z = aggv * n_in + b_ref[...]
        o_ref[...] = jnp.maximum(z, 0.0) * n_out

    return pl.pallas_call(
        body,
        grid=(N // R,),
        in_specs=[
            pl.BlockSpec((2, R, F_H), lambda i: (0, i, 0)),
            pl.BlockSpec((2, 2, R, 16), lambda i: (0, 0, i, 0)),
            pl.BlockSpec((1, F_H), lambda i: (0, 0)),
        ],
        out_specs=pl.BlockSpec((R, F_H), lambda i: (i, 0)),
        out_shape=jax.ShapeDtypeStruct((N, F_H), jnp.float32),
    )(agg_parts, degs, b1)


def _tc_out(agg_parts, degs, W2, b2):
    def body(a_ref, deg_ref, w_ref, b_ref, o_ref):
        dg = deg_ref[...]
        n_in = _norms(dg, 1)
        aggv = (a_ref[0] + a_ref[1]) * n_in
        o_ref[...] = jnp.dot(aggv, w_ref[...],
                             preferred_element_type=jnp.float32) + b_ref[...]

    return pl.pallas_call(
        body,
        grid=(N // R,),
        in_specs=[
            pl.BlockSpec((2, R, F_H), lambda i: (0, i, 0)),
            pl.BlockSpec((2, 2, R, 16), lambda i: (0, 0, i, 0)),
            pl.BlockSpec((F_H, F_C), lambda i: (0, 0)),
            pl.BlockSpec((1, F_C), lambda i: (0, 0)),
        ],
        out_specs=pl.BlockSpec((R, F_C), lambda i: (i, 0)),
        out_shape=jax.ShapeDtypeStruct((N, F_C), jnp.float32),
    )(agg_parts, degs, W2, b2)


def kernel(in_feat, edge_index, W1, b1, W2, b2):
    src = edge_index[0].astype(jnp.int32)
    dst = edge_index[1].astype(jnp.int32)
    z16 = jnp.zeros((NS, RPS, 16), jnp.float32)
    z128 = jnp.zeros((NS, RPS, F_H), jnp.float32)
    ones16 = jnp.ones((K, 16), jnp.float32)

    degs = _make_deg()(src, dst, z16, ones16).reshape(NC, 2, N, 16)
    h1 = _tc_h1(in_feat, degs, W1)
    agg1 = _make_agg(F_H)(h1, src, dst, z128).reshape(NC, N, F_H)
    u = _tc_mid(agg1, degs, b1.reshape(1, F_H))
    agg2 = _make_agg(F_H)(u, src, dst, z128).reshape(NC, N, F_H)
    return _tc_out(agg2, degs, W2, b2.reshape(1, F_C))


# speedup vs baseline: 1.0953x; 1.0953x over previous
import functools

import jax
import jax.numpy as jnp
from jax import lax
from jax.experimental import pallas as pl
from jax.experimental.pallas import tpu as pltpu
from jax.experimental.pallas import tpu_sc as plsc

N = 10000
E = 320000
F_IN = 128
F_H = 128
F_C = 40
F_CP = 48

NC, NS = 2, 16
NW = NC * NS
EPW = E // NW
K = 80
NCHUNK = EPW // K
KA = 128
NF = EPW // KA
KT = EPW - NF * KA
RPS = N // NS

_MESH = dict(core_axis_name="c", subcore_axis_name="s",
             num_cores=NC, num_subcores=NS)


@functools.cache
def _make_agg(D):
    @functools.partial(
        pl.kernel,
        out_type=jax.ShapeDtypeStruct((NC, NS, RPS, D), jnp.float32),
        mesh=plsc.VectorSubcoreMesh(**_MESH),
        scratch_types=[
            [pltpu.VMEM((KA, D), jnp.float32)] * 3,
            [pltpu.VMEM((KA,), jnp.int32)] * 3,
            [pltpu.VMEM((KA,), jnp.int32)] * 3,
            pltpu.VMEM_SHARED((N, D), jnp.float32),
            [pltpu.SemaphoreType.DMA] * 3,
            [pltpu.SemaphoreType.DMA] * 3,
            [pltpu.SemaphoreType.DMA] * 3,
        ],
    )
    def agg(h_hbm, src_hbm, dst_hbm, z_hbm, out_hbm,
            rows, isv, idv, acc, isem, gsem, ssem):
        c = lax.axis_index("c")
        s = lax.axis_index("s")
        wid = s * NC + c
        base = wid * EPW

        def fire_i(t, b, size=KA):
            pltpu.async_copy(src_hbm.at[pl.ds(base + t * KA, size)],
                             isv[b].at[pl.ds(0, size)], isem[b])
            pltpu.async_copy(dst_hbm.at[pl.ds(base + t * KA, size)],
                             idv[b].at[pl.ds(0, size)], isem[b])

        def wait_i(t, b, size=KA):
            pltpu.make_async_copy(src_hbm.at[pl.ds(base + t * KA, size)],
                                  isv[b].at[pl.ds(0, size)], isem[b]).wait()
            pltpu.make_async_copy(dst_hbm.at[pl.ds(base + t * KA, size)],
                                  idv[b].at[pl.ds(0, size)], isem[b]).wait()

        def fire_g(b):
            pltpu.async_copy(h_hbm.at[isv[b]], rows[b], gsem[b])

        def wait_g(b):
            pltpu.make_async_copy(h_hbm.at[isv[b]], rows[b], gsem[b]).wait()

        def fire_s(b):
            pltpu.async_copy(rows[b], acc.at[idv[b]], ssem[b], add=True)

        def wait_s(b):
            pltpu.make_async_copy(rows[b], acc.at[idv[b]], ssem[b]).wait()

        fire_i(0, 0)
        fire_i(1, 1)
        pltpu.sync_copy(z_hbm.at[s], acc.at[pl.ds(s * RPS, RPS)])
        wait_i(0, 0)
        fire_g(0)
        plsc.subcore_barrier()

        def step(q, carry):
            for b in range(3):
                t = 3 * q + b
                bp = (b + 2) % 3
                bn = (b + 1) % 3
                if b == 0:
                    @pl.when(q > 0)
                    def _():
                        wait_s(bp)
                else:
                    wait_s(bp)
                fire_i(t + 2, bp)
                wait_g(b)
                fire_s(b)
                wait_i(t + 1, bn)
                fire_g(bn)
            return carry

        lax.fori_loop(0, NF // 3 - 1, step, 0)
        for t in (NF - 3, NF - 2, NF - 1):
            b, bp, bn = t % 3, (t + 2) % 3, (t + 1) % 3
            wait_s(bp)
            if t == NF - 3:
                fire_i(t + 2, bp)
            elif t == NF - 2:
                fire_i(NF, bp, size=KT)
            wait_g(b)
            fire_s(b)
            if t < NF - 1:
                wait_i(t + 1, bn)
                fire_g(bn)
        bt = NF % 3
        wait_i(NF, bt, size=KT)
        d_t = pltpu.async_copy(h_hbm.at[isv[bt].at[pl.ds(0, KT)]],
                               rows[bt].at[pl.ds(0, KT), :], gsem[bt])
        wait_s((NF - 1) % 3)
        d_t.wait()
        pltpu.sync_copy(rows[bt].at[pl.ds(0, KT), :],
                        acc.at[idv[bt].at[pl.ds(0, KT)]], add=True)
        plsc.subcore_barrier()
        pltpu.sync_copy(acc.at[pl.ds(s * RPS, RPS)], out_hbm.at[c, s])

    return agg


@functools.cache
def _make_deg():
    @functools.partial(
        pl.kernel,
        out_type=jax.ShapeDtypeStruct((NC, 2, NS, RPS, 16), jnp.float32),
        mesh=plsc.VectorSubcoreMesh(**_MESH),
        scratch_types=[
            pltpu.VMEM((EPW,), jnp.int32),
            pltpu.VMEM((EPW,), jnp.int32),
            pltpu.VMEM((K, 16), jnp.float32),
            pltpu.VMEM_SHARED((N, 16), jnp.float32),
            pltpu.VMEM_SHARED((N, 16), jnp.float32),
            pltpu.SemaphoreType.DMA,
        ],
        compiler_params=pltpu.CompilerParams(use_tc_tiling_on_sc=False),
    )
    def _deg(src_hbm, dst_hbm, z_hbm, ones_hbm, out_hbm,
             srcv, dstv, ones_v, acc_o, acc_i, sem):
        c = lax.axis_index("c")
        s = lax.axis_index("s")
        wid = s * NC + c
        base = wid * EPW
        pltpu.sync_copy(src_hbm.at[pl.ds(base, EPW)], srcv)
        pltpu.sync_copy(dst_hbm.at[pl.ds(base, EPW)], dstv)
        pltpu.sync_copy(ones_hbm, ones_v)
        pltpu.sync_copy(z_hbm.at[s], acc_o.at[pl.ds(s * RPS, RPS)])
        pltpu.sync_copy(z_hbm.at[s], acc_i.at[pl.ds(s * RPS, RPS)])
        plsc.subcore_barrier()

        def fire(t, carry):
            pltpu.async_copy(ones_v, acc_o.at[srcv.at[pl.ds(t * K, K)]], sem,
                             add=True)
            pltpu.async_copy(ones_v, acc_i.at[dstv.at[pl.ds(t * K, K)]], sem,
                             add=True)
            return carry

        lax.fori_loop(0, NCHUNK, fire, 0)

        def drain(t, carry):
            pltpu.make_async_copy(ones_v, acc_o.at[srcv.at[pl.ds(t * K, K)]],
                                  sem).wait()
            pltpu.make_async_copy(ones_v, acc_i.at[dstv.at[pl.ds(t * K, K)]],
                                  sem).wait()
            return carry

        lax.fori_loop(0, NCHUNK, drain, 0)
        plsc.subcore_barrier()
        pltpu.sync_copy(acc_o.at[pl.ds(s * RPS, RPS)], out_hbm.at[c, 0, s])
        pltpu.sync_copy(acc_i.at[pl.ds(s * RPS, RPS)], out_hbm.at[c, 1, s])

    return _deg


R = 400


def _norms(dg, which):
    d = dg[0, which, :, 0:1] + dg[1, which, :, 0:1]
    return lax.rsqrt(jnp.maximum(d, 1.0))


def _tc_h1(x, degs, W1):
    def body(x_ref, deg_ref, w_ref, o_ref):
        dg = deg_ref[...]
        n_out = _norms(dg, 0)
        o_ref[...] = jnp.dot(x_ref[...] * n_out, w_ref[...],
                             preferred_element_type=jnp.float32)

    return pl.pallas_call(
        body,
        grid=(N // R,),
        in_specs=[
            pl.BlockSpec((R, F_IN), lambda i: (i, 0)),
            pl.BlockSpec((2, 2, R, 16), lambda i: (0, 0, i, 0)),
            pl.BlockSpec((F_IN, F_H), lambda i: (0, 0)),
        ],
        out_specs=pl.BlockSpec((R, F_H), lambda i: (i, 0)),
        out_shape=jax.ShapeDtypeStruct((N, F_H), jnp.float32),
    )(x, degs, W1)


def _tc_mid(agg_parts, degs, b1):
    def body(a_ref, deg_ref, b_ref, o_ref):
        dg = deg_ref[...]
        n_out = _norms(dg, 0)
        n_in = _norms(dg, 1)
        aggv = a_ref[0] + a_ref[1]
        z = aggv * n_in + b_ref[...]
        o_ref[...] = jnp.maximum(z, 0.0) * n_out

    return pl.pallas_call(
        body,
        grid=(N // R,),
        in_specs=[
            pl.BlockSpec((2, R, F_H), lambda i: (0, i, 0)),
            pl.BlockSpec((2, 2, R, 16), lambda i: (0, 0, i, 0)),
            pl.BlockSpec((1, F_H), lambda i: (0, 0)),
        ],
        out_specs=pl.BlockSpec((R, F_H), lambda i: (i, 0)),
        out_shape=jax.ShapeDtypeStruct((N, F_H), jnp.float32),
    )(agg_parts, degs, b1)


def _tc_out(agg_parts, degs, W2, b2):
    def body(a_ref, deg_ref, w_ref, b_ref, o_ref):
        dg = deg_ref[...]
        n_in = _norms(dg, 1)
        aggv = (a_ref[0] + a_ref[1]) * n_in
        o_ref[...] = jnp.dot(aggv, w_ref[...],
                             preferred_element_type=jnp.float32) + b_ref[...]

    return pl.pallas_call(
        body,
        grid=(N // R,),
        in_specs=[
            pl.BlockSpec((2, R, F_H), lambda i: (0, i, 0)),
            pl.BlockSpec((2, 2, R, 16), lambda i: (0, 0, i, 0)),
            pl.BlockSpec((F_H, F_C), lambda i: (0, 0)),
            pl.BlockSpec((1, F_C), lambda i: (0, 0)),
        ],
        out_specs=pl.BlockSpec((R, F_C), lambda i: (i, 0)),
        out_shape=jax.ShapeDtypeStruct((N, F_C), jnp.float32),
    )(agg_parts, degs, W2, b2)


def kernel(in_feat, edge_index, W1, b1, W2, b2):
    src = edge_index[0].astype(jnp.int32)
    dst = edge_index[1].astype(jnp.int32)
    z16 = jnp.zeros((NS, RPS, 16), jnp.float32)
    z128 = jnp.zeros((NS, RPS, F_H), jnp.float32)
    ones16 = jnp.ones((K, 16), jnp.float32)

    degs = _make_deg()(src, dst, z16, ones16).reshape(NC, 2, N, 16)
    h1 = _tc_h1(in_feat, degs, W1)
    agg1 = _make_agg(F_H)(h1, src, dst, z128).reshape(NC, N, F_H)
    u = _tc_mid(agg1, degs, b1.reshape(1, F_H))
    agg2 = _make_agg(F_H)(u, src, dst, z128).reshape(NC, N, F_H)
    return _tc_out(agg2, degs, W2, b2.reshape(1, F_C))
